# Initial kernel scaffold; baseline (speedup 1.0000x reference)
import jax
import jax.numpy as jnp
from jax.experimental import pallas as pl


def kernel(input, target, proba):
    raise NotImplementedError("write your pallas kernel here")


# speedup vs baseline: 2.4767x; 2.4767x over previous
import functools
import math

import jax
import jax.numpy as jnp
from jax.experimental import pallas as pl
from jax.experimental.pallas import tpu as pltpu

NB_ANGLES = 100
B = 32
N = 8192
SUB = 64
A_BLK = 2
ROWS = SUB * 2 * A_BLK


def _bitonic_sort_blocks(y):
    s_iota = jax.lax.broadcasted_iota(jnp.int32, y.shape, 0) & (SUB - 1)
    l_iota = jax.lax.broadcasted_iota(jnp.int32, y.shape, 1)

    def bit_is_zero(v):
        if v < 128:
            return (l_iota & v) == 0
        return (s_iota & (v >> 7)) == 0

    for st in range(1, 14):
        k = 1 << st
        kb = bit_is_zero(k)
        for e in range(st - 1, -1, -1):
            j = 1 << e
            if j >= 128:
                m = j >> 7
                up = pltpu.roll(y, y.shape[0] - m, 0)
                dn = pltpu.roll(y, m, 0)
            else:
                up = pltpu.roll(y, 128 - j, 1)
                dn = pltpu.roll(y, j, 1)
            jb = bit_is_zero(j)
            partner = jnp.where(jb, up, dn)
            mn = jnp.minimum(y, partner)
            mx = jnp.maximum(y, partner)
            y = jnp.where(jb == kb, mn, mx)
    return y


def _swd_kernel(ang_ref, proba_ref, x_ref, t_ref, out_ref, acc_ref):
    b = pl.program_id(0)
    ab = pl.program_id(1)
    na = pl.num_programs(1)

    pieces = []
    for a in range(A_BLK):
        aidx = ab * A_BLK + a
        a0 = ang_ref[0, aidx]
        a1 = ang_ref[1, aidx]
        a2 = ang_ref[2, aidx]
        for ref in (x_ref, t_ref):
            proj = ref[0, 0] * a0 + ref[0, 1] * a1 + ref[0, 2] * a2
            pieces.append(proj)
    y = jnp.concatenate(pieces, axis=0)

    y = _bitonic_sort_blocks(y)

    ssum = jnp.float32(0.0)
    for a in range(A_BLK):
        lo = y[2 * a * SUB:(2 * a + 1) * SUB]
        hi = y[(2 * a + 1) * SUB:(2 * a + 2) * SUB]
        ssum += jnp.sum(jnp.abs(lo - hi))

    pw = proba_ref[b]

    @pl.when(jnp.logical_and(b == 0, ab == 0))
    def _():
        acc_ref[0] = 0.0
        acc_ref[1] = 0.0

    acc_ref[0] += pw * ssum * (1.0 / (N * NB_ANGLES))

    @pl.when(ab == 0)
    def _():
        acc_ref[1] += pw

    @pl.when(jnp.logical_and(b == B - 1, ab == na - 1))
    def _():
        out_ref[0] = acc_ref[0] / acc_ref[1]


@jax.jit
def kernel(input, target, proba):
    akey = jax.random.key(42)
    angles = jax.random.uniform(akey, (1, 1, NB_ANGLES, 3), dtype=jnp.float32) * 2.0 - 1.0
    angles = angles / jnp.linalg.norm(angles, axis=-1, keepdims=True)
    ang = angles.reshape(NB_ANGLES, 3).T

    x = input.transpose(0, 2, 1).reshape(B, 3, SUB, 128)
    t = target.transpose(0, 2, 1).reshape(B, 3, SUB, 128)

    grid = (B, NB_ANGLES // A_BLK)
    out = pl.pallas_call(
        _swd_kernel,
        grid=grid,
        in_specs=[
            pl.BlockSpec(memory_space=pltpu.SMEM),
            pl.BlockSpec(memory_space=pltpu.SMEM),
            pl.BlockSpec((1, 3, SUB, 128), lambda b, a: (b, 0, 0, 0)),
            pl.BlockSpec((1, 3, SUB, 128), lambda b, a: (b, 0, 0, 0)),
        ],
        out_specs=pl.BlockSpec(memory_space=pltpu.SMEM),
        out_shape=jax.ShapeDtypeStruct((1,), jnp.float32),
        scratch_shapes=[pltpu.SMEM((2,), jnp.float32)],
    )(ang, proba, x, t)
    return out.reshape(())


# speedup vs baseline: 3.5163x; 1.4197x over previous
import jax
import jax.numpy as jnp
from jax.experimental import pallas as pl
from jax.experimental.pallas import tpu as pltpu

NB_ANGLES = 100
B = 32
N = 8192
SUB = 64
A_BLK = 4
ROWS = 2 * SUB


def _bitonic_sort_chains(ys):
    shape = ys[0].shape
    s_iota = jax.lax.broadcasted_iota(jnp.int32, shape, 0)
    l_iota = jax.lax.broadcasted_iota(jnp.int32, shape, 1)
    idx = ((s_iota & (SUB - 1)) << 7) | l_iota

    one = jnp.float32(1.0)
    neg = jnp.float32(-1.0)

    for st in range(1, 14):
        k = 1 << st
        if k < N:
            sgn = jnp.where((idx & k) == 0, one, neg)
            ws = [y * sgn for y in ys]
        else:
            ws = ys
        for e in range(st - 1, -1, -1):
            j = 1 << e
            jb = (idx & j) == 0
            for c in range(len(ws)):
                w = ws[c]
                if j >= 128:
                    m = j >> 7
                    up = pltpu.roll(w, ROWS - m, 0)
                    dn = pltpu.roll(w, m, 0)
                else:
                    up = pltpu.roll(w, 128 - j, 1)
                    dn = pltpu.roll(w, j, 1)
                partner = jnp.where(jb, up, dn)
                ws[c] = jnp.where(jb, jnp.minimum(w, partner),
                                  jnp.maximum(w, partner))
        if k < N:
            ys = [w * sgn for w in ws]
        else:
            ys = ws
    return ys


def _swd_kernel(ang_ref, proba_ref, x_ref, t_ref, out_ref, acc_ref):
    b = pl.program_id(0)
    ab = pl.program_id(1)
    na = pl.num_programs(1)

    ys = []
    for a in range(A_BLK):
        aidx = ab * A_BLK + a
        a0 = ang_ref[0, aidx]
        a1 = ang_ref[1, aidx]
        a2 = ang_ref[2, aidx]
        pin = x_ref[0, 0] * a0 + x_ref[0, 1] * a1 + x_ref[0, 2] * a2
        ptg = t_ref[0, 0] * a0 + t_ref[0, 1] * a1 + t_ref[0, 2] * a2
        ys.append(jnp.concatenate([pin, ptg], axis=0))

    ys = _bitonic_sort_chains(ys)

    ssum = jnp.float32(0.0)
    for y in ys:
        ssum += jnp.sum(jnp.abs(y[:SUB] - y[SUB:]))

    pw = proba_ref[b]

    @pl.when(jnp.logical_and(b == 0, ab == 0))
    def _():
        acc_ref[0] = 0.0
        acc_ref[1] = 0.0

    acc_ref[0] += pw * ssum * (1.0 / (N * NB_ANGLES))

    @pl.when(ab == 0)
    def _():
        acc_ref[1] += pw

    @pl.when(jnp.logical_and(b == B - 1, ab == na - 1))
    def _():
        out_ref[0] = acc_ref[0] / acc_ref[1]


@jax.jit
def kernel(input, target, proba):
    akey = jax.random.key(42)
    angles = jax.random.uniform(akey, (1, 1, NB_ANGLES, 3), dtype=jnp.float32) * 2.0 - 1.0
    angles = angles / jnp.linalg.norm(angles, axis=-1, keepdims=True)
    ang = angles.reshape(NB_ANGLES, 3).T

    x = input.transpose(0, 2, 1).reshape(B, 3, SUB, 128)
    t = target.transpose(0, 2, 1).reshape(B, 3, SUB, 128)

    grid = (B, NB_ANGLES // A_BLK)
    out = pl.pallas_call(
        _swd_kernel,
        grid=grid,
        in_specs=[
            pl.BlockSpec(memory_space=pltpu.SMEM),
            pl.BlockSpec(memory_space=pltpu.SMEM),
            pl.BlockSpec((1, 3, SUB, 128), lambda b, a: (b, 0, 0, 0)),
            pl.BlockSpec((1, 3, SUB, 128), lambda b, a: (b, 0, 0, 0)),
        ],
        out_specs=pl.BlockSpec(memory_space=pltpu.SMEM),
        out_shape=jax.ShapeDtypeStruct((1,), jnp.float32),
        scratch_shapes=[pltpu.SMEM((2,), jnp.float32)],
    )(ang, proba, x, t)
    return out.reshape(())
